# baseline (speedup 1.0000x reference)
import functools

import jax
import jax.numpy as jnp
from jax import lax
from jax.experimental import pallas as pl
from jax.experimental.pallas import tpu as pltpu

N_ENT = 50000
N_TOT = 100000
D = 128
N_EDGES = 600000
BATCH = 1024

GATE_BLK = 2000


def _gate_body(pre_ref, collab_ref, kg_ref, wb_ref, dual_ref):
    collab = collab_ref[...]
    kg = kg_ref[...]
    z = pre_ref[...] + jnp.dot(collab, wb_ref[...],
                               preferred_element_type=jnp.float32)
    g = jax.nn.sigmoid(z)
    dual_ref[...] = g * kg + (1.0 - g) * collab


def _gate(pre_a, collab, kg, W_b_t):
    grid = (N_ENT // GATE_BLK,)
    return pl.pallas_call(
        _gate_body,
        grid=grid,
        in_specs=[
            pl.BlockSpec((GATE_BLK, D), lambda i: (i, 0)),
            pl.BlockSpec((GATE_BLK, D), lambda i: (i, 0)),
            pl.BlockSpec((GATE_BLK, D), lambda i: (i, 0)),
            pl.BlockSpec((D, D), lambda i: (0, 0)),
        ],
        out_specs=pl.BlockSpec((GATE_BLK, D), lambda i: (i, 0)),
        out_shape=jax.ShapeDtypeStruct((N_ENT, D), jnp.float32),
    )(pre_a, collab, kg, W_b_t)


def _prea_body(ent_ref, wa_ref, out_ref):
    out_ref[...] = jnp.dot(ent_ref[...], wa_ref[...],
                           preferred_element_type=jnp.float32)


def _pre_a(e_entities, W_a_t):
    grid = (N_ENT // GATE_BLK,)
    return pl.pallas_call(
        _prea_body,
        grid=grid,
        in_specs=[
            pl.BlockSpec((GATE_BLK, D), lambda i: (i, 0)),
            pl.BlockSpec((D, D), lambda i: (0, 0)),
        ],
        out_specs=pl.BlockSpec((GATE_BLK, D), lambda i: (i, 0)),
        out_shape=jax.ShapeDtypeStruct((N_ENT, D), jnp.float32),
    )(e_entities, W_a_t)


def _score_body(u_ref, it_ref, out_ref):
    out_ref[...] = jnp.dot(u_ref[...], it_ref[...].T,
                           preferred_element_type=jnp.float32)


def _scores(user_embed, item_embed):
    return pl.pallas_call(
        _score_body,
        out_shape=jax.ShapeDtypeStruct((BATCH, BATCH), jnp.float32),
    )(user_embed, item_embed)


def _spmm(rows, cols, vals, x):
    gathered = vals[:, None] * jnp.take(x, cols, axis=0)
    return jax.ops.segment_sum(gathered, rows, num_segments=N_TOT)


def kernel(entity_user_embed, W_a, W_b, A_vals, A_rows, A_cols, user_ids, item_ids):
    e_entities = entity_user_embed[:N_ENT]
    e_users = entity_user_embed[N_ENT:]
    W_a_t = W_a.T
    W_b_t = W_b.T

    pre_a = _pre_a(e_entities, W_a_t)

    x = entity_user_embed
    collab_sum = e_entities
    users_sum = e_users
    for layer in range(3):
        y = _spmm(A_rows, A_cols, A_vals, x)
        collab = y[:N_ENT]
        users = y[N_ENT:]
        collab_sum = collab_sum + collab
        users_sum = users_sum + users
        if layer < 2:
            dual = _gate(pre_a, collab, e_entities, W_b_t)
            x = jnp.concatenate([dual, users], axis=0)

    all_embed = jnp.concatenate([collab_sum, users_sum], axis=0)
    user_embed = jnp.take(all_embed, user_ids, axis=0)
    item_embed = jnp.take(all_embed, item_ids, axis=0)
    return _scores(user_embed, item_embed)


# speedup vs baseline: 2.2891x; 2.2891x over previous
import functools

import jax
import jax.numpy as jnp
from jax import lax
from jax.experimental import pallas as pl
from jax.experimental.pallas import tpu as pltpu
from jax.experimental.pallas import tpu_sc as plsc

N_ENT = 50000
N_TOT = 100000
D = 128
N_EDGES = 600000
BATCH = 1024

GATE_BLK = 2000

NC = 2
NS = 16
B = 128
PASS_R = 10000
NPASS = 5
EPAD = N_EDGES + NS * B
TRASH = 10000
ACC_ROWS = 10112
WSLICE = 632
W15 = 520


def _spmm_body(x_hbm, scols_hbm, srows_hbm, svals_hbm, starts_hbm, nb_hbm,
               out_hbm, acc, colb, rowb, valb, lidxb, rowsbuf,
               startv, nbv, gsem):
    c = lax.axis_index("c")
    s = lax.axis_index("s")

    def pass_body(p, carry):
        pidx = c * NPASS + p
        base = c * N_ENT + p * PASS_R
        poff = pl.multiple_of(pidx * 16, 8)
        pltpu.sync_copy(starts_hbm.at[pl.ds(poff, 16)], startv)
        pltpu.sync_copy(nb_hbm.at[pl.ds(poff, 16)], nbv)
        start = jnp.max(startv[...])
        nb = jnp.max(nbv[...])

        @functools.partial(lax.fori_loop, 0, 128, init_val=0)
        def _(i, zcarry):
            iv = jnp.full((16,), i, dtype=jnp.int32)
            for d in range(D // 16):
                ci = d * 16 + lax.iota(jnp.int32, 16)
                plsc.store_scatter(rowsbuf, [iv, ci],
                                   jnp.zeros((16,), jnp.float32))
            return zcarry

        zsrc = rowsbuf.at[pl.ds(0, 128)]
        zlo = s * WSLICE
        for k in range(WSLICE // 128):
            pltpu.sync_copy(zsrc, acc.at[pl.ds(zlo + k * 128, 128)])
        pltpu.sync_copy(rowsbuf.at[pl.ds(0, WSLICE % 128)],
                        acc.at[pl.ds(zlo + (WSLICE // 128) * 128,
                                     WSLICE % 128)])

        plsc.subcore_barrier()

        tstart = start + s * nb * B

        def batch_body(i, bcarry):
            off = pl.multiple_of(tstart + i * B, 8)
            pltpu.sync_copy(scols_hbm.at[pl.ds(off, B)], colb)
            pltpu.sync_copy(srows_hbm.at[pl.ds(off, B)], rowb)
            pltpu.sync_copy(svals_hbm.at[pl.ds(off, B)], valb)
            pltpu.async_copy(x_hbm.at[colb], rowsbuf, gsem).wait()

            for j8 in range(B // 16):
                r16 = rowb[pl.ds(j8 * 16, 16)]
                l16 = r16 - base
                bad = (l16 < 0) | (l16 >= PASS_R)
                lidxb[pl.ds(j8 * 16, 16)] = jnp.where(bad, TRASH, l16)

            @functools.partial(lax.fori_loop, 0, B, init_val=0)
            def _(j, scarry):
                jv = jnp.full((16,), j, dtype=jnp.int32)
                v = plsc.load_gather(valb, [jv])
                for d in range(D // 16):
                    ci = d * 16 + lax.iota(jnp.int32, 16)
                    xv = plsc.load_gather(rowsbuf, [jv, ci])
                    plsc.store_scatter(rowsbuf, [jv, ci], xv * v)
                return scarry

            pltpu.sync_copy(rowsbuf, acc.at[lidxb], add=True)
            return bcarry

        lax.fori_loop(0, nb, batch_body, 0)

        plsc.subcore_barrier()

        @pl.when(s < NS - 1)
        def _():
            pltpu.sync_copy(acc.at[pl.ds(zlo, WSLICE)],
                            out_hbm.at[pl.ds(base + zlo, WSLICE)])

        @pl.when(s == NS - 1)
        def _():
            pltpu.sync_copy(acc.at[pl.ds(zlo, W15)],
                            out_hbm.at[pl.ds(base + zlo, W15)])

        return carry

    lax.fori_loop(0, NPASS, pass_body, 0)


_spmm_call = pl.kernel(
    _spmm_body,
    out_type=jax.ShapeDtypeStruct((N_TOT, D), jnp.float32),
    mesh=plsc.VectorSubcoreMesh(core_axis_name="c", subcore_axis_name="s",
                                num_cores=NC, num_subcores=NS),
    compiler_params=pltpu.CompilerParams(needs_layout_passes=False),
    scratch_types=[
        pltpu.VMEM_SHARED((ACC_ROWS, D), jnp.float32),
        pltpu.VMEM((B,), jnp.int32),
        pltpu.VMEM((B,), jnp.int32),
        pltpu.VMEM((B,), jnp.float32),
        pltpu.VMEM((B,), jnp.int32),
        pltpu.VMEM((B, D), jnp.float32),
        pltpu.VMEM((16,), jnp.int32),
        pltpu.VMEM((16,), jnp.int32),
        pltpu.SemaphoreType.DMA,
    ],
)


def _sort_edges(rows, cols, vals):
    order = jnp.argsort(rows)
    srows = jnp.take(rows, order)
    scols = jnp.take(cols, order)
    svals = jnp.take(vals, order)
    pad = EPAD - N_EDGES
    srows_p = jnp.concatenate(
        [srows, jnp.full((pad,), jnp.int32(1 << 20))])
    scols_p = jnp.concatenate([scols, jnp.zeros((pad,), jnp.int32)])
    svals_p = jnp.concatenate([svals, jnp.zeros((pad,), jnp.float32)])

    cuts = [min(c * N_ENT + p * PASS_R, (c + 1) * N_ENT)
            for c in range(NC) for p in range(NPASS)] + [N_TOT]
    bounds = jnp.searchsorted(srows, jnp.array(cuts, dtype=jnp.int32))
    starts = (bounds[:-1] // 8) * 8
    lens = bounds[1:] - starts
    nb = (lens + NS * B - 1) // (NS * B)
    starts_b = jnp.broadcast_to(
        starts[:, None], (NC * NPASS, 16)).astype(jnp.int32).reshape(-1)
    nb_b = jnp.broadcast_to(
        nb[:, None], (NC * NPASS, 16)).astype(jnp.int32).reshape(-1)
    return srows_p, scols_p, svals_p, starts_b, nb_b


def _gate_body(pre_ref, collab_ref, kg_ref, wb_ref, dual_ref):
    collab = collab_ref[...]
    kg = kg_ref[...]
    z = pre_ref[...] + jnp.dot(collab, wb_ref[...],
                               preferred_element_type=jnp.float32)
    g = jax.nn.sigmoid(z)
    dual_ref[...] = g * kg + (1.0 - g) * collab


def _gate(pre_a, collab, kg, W_b_t):
    grid = (N_ENT // GATE_BLK,)
    return pl.pallas_call(
        _gate_body,
        grid=grid,
        in_specs=[
            pl.BlockSpec((GATE_BLK, D), lambda i: (i, 0)),
            pl.BlockSpec((GATE_BLK, D), lambda i: (i, 0)),
            pl.BlockSpec((GATE_BLK, D), lambda i: (i, 0)),
            pl.BlockSpec((D, D), lambda i: (0, 0)),
        ],
        out_specs=pl.BlockSpec((GATE_BLK, D), lambda i: (i, 0)),
        out_shape=jax.ShapeDtypeStruct((N_ENT, D), jnp.float32),
    )(pre_a, collab, kg, W_b_t)


def _prea_body(ent_ref, wa_ref, out_ref):
    out_ref[...] = jnp.dot(ent_ref[...], wa_ref[...],
                           preferred_element_type=jnp.float32)


def _pre_a(e_entities, W_a_t):
    grid = (N_ENT // GATE_BLK,)
    return pl.pallas_call(
        _prea_body,
        grid=grid,
        in_specs=[
            pl.BlockSpec((GATE_BLK, D), lambda i: (i, 0)),
            pl.BlockSpec((D, D), lambda i: (0, 0)),
        ],
        out_specs=pl.BlockSpec((GATE_BLK, D), lambda i: (i, 0)),
        out_shape=jax.ShapeDtypeStruct((N_ENT, D), jnp.float32),
    )(e_entities, W_a_t)


def _score_body(u_ref, it_ref, out_ref):
    out_ref[...] = jnp.dot(u_ref[...], it_ref[...].T,
                           preferred_element_type=jnp.float32)


def _scores(user_embed, item_embed):
    return pl.pallas_call(
        _score_body,
        out_shape=jax.ShapeDtypeStruct((BATCH, BATCH), jnp.float32),
    )(user_embed, item_embed)


def kernel(entity_user_embed, W_a, W_b, A_vals, A_rows, A_cols, user_ids, item_ids):
    e_entities = entity_user_embed[:N_ENT]
    e_users = entity_user_embed[N_ENT:]
    W_a_t = W_a.T
    W_b_t = W_b.T

    pre_a = _pre_a(e_entities, W_a_t)
    srows_p, scols_p, svals_p, starts_b, nb_b = _sort_edges(
        A_rows, A_cols, A_vals)

    x = entity_user_embed
    collab_sum = e_entities
    users_sum = e_users
    for layer in range(3):
        y = _spmm_call(x, scols_p, srows_p, svals_p, starts_b, nb_b)
        collab = y[:N_ENT]
        users = y[N_ENT:]
        collab_sum = collab_sum + collab
        users_sum = users_sum + users
        if layer < 2:
            dual = _gate(pre_a, collab, e_entities, W_b_t)
            x = jnp.concatenate([dual, users], axis=0)

    all_embed = jnp.concatenate([collab_sum, users_sum], axis=0)
    user_embed = jnp.take(all_embed, user_ids, axis=0)
    item_embed = jnp.take(all_embed, item_ids, axis=0)
    return _scores(user_embed, item_embed)


# speedup vs baseline: 2.4062x; 1.0512x over previous
import functools

import jax
import jax.numpy as jnp
from jax import lax
from jax.experimental import pallas as pl
from jax.experimental.pallas import tpu as pltpu
from jax.experimental.pallas import tpu_sc as plsc

N_ENT = 50000
N_TOT = 100000
D = 128
N_EDGES = 600000
BATCH = 1024

GATE_BLK = 2000

NC = 2
NS = 16
B = 128
PASS_R = 10000
NPASS = 5
EPAD = N_EDGES + NS * B
TRASH = 10000
ACC_ROWS = 10112
WSLICE = 632
W15 = 520


def _iota16():
    return lax.iota(jnp.int32, 16)


def _extract_colb(pbuf, colb):
    for j8 in range(B // 16):
        j16 = j8 * 16 + _iota16()
        c16 = plsc.load_gather(pbuf, [j16 * 3])
        colb[pl.ds(j8 * 16, 16)] = c16


def _spmm_body(x_hbm, pk_hbm, starts_hbm, nb_hbm, out_hbm,
               acc, pbuf0, pbuf1, colb0, colb1, lidxb, rowsbuf0, rowsbuf1,
               startv, nbv, isem0, isem1, gsem0, gsem1):
    c = lax.axis_index("c")
    s = lax.axis_index("s")
    pbufs = (pbuf0, pbuf1)
    colbs = (colb0, colb1)
    rowsbufs = (rowsbuf0, rowsbuf1)
    isems = (isem0, isem1)
    gsems = (gsem0, gsem1)

    def pass_body(p, carry):
        pidx = c * NPASS + p
        base = c * N_ENT + p * PASS_R
        poff = pl.multiple_of(pidx * 16, 8)
        pltpu.sync_copy(starts_hbm.at[pl.ds(poff, 16)], startv)
        pltpu.sync_copy(nb_hbm.at[pl.ds(poff, 16)], nbv)
        start = jnp.max(startv[...])
        nb = jnp.max(nbv[...])
        tstart = start + s * nb * B

        def pk_slice(k):
            return pk_hbm.at[pl.ds(pl.multiple_of((tstart + k * B) * 3, 8),
                                   3 * B)]

        @functools.partial(lax.fori_loop, 0, 128, init_val=0)
        def _(i, zcarry):
            iv = jnp.full((16,), i, dtype=jnp.int32)
            for d in range(D // 16):
                ci = d * 16 + _iota16()
                plsc.store_scatter(rowsbuf0, [iv, ci],
                                   jnp.zeros((16,), jnp.float32))
            return zcarry

        zlo = s * WSLICE
        for k in range(WSLICE // 128):
            pltpu.sync_copy(rowsbuf0.at[pl.ds(0, 128)],
                            acc.at[pl.ds(zlo + k * 128, 128)])
        pltpu.sync_copy(rowsbuf0.at[pl.ds(0, WSLICE % 128)],
                        acc.at[pl.ds(zlo + (WSLICE // 128) * 128,
                                     WSLICE % 128)])

        plsc.subcore_barrier()

        @pl.when(nb > 0)
        def _():
            pltpu.async_copy(pk_slice(0), pbuf0, isem0)

            @pl.when(nb > 1)
            def _():
                pltpu.async_copy(pk_slice(1), pbuf1, isem1)

            pltpu.make_async_copy(pk_slice(0), pbuf0, isem0).wait()
            _extract_colb(pbuf0, colb0)
            pltpu.async_copy(x_hbm.at[colb0], rowsbuf0, gsem0)

        def slot(i, b):
            pbuf, colb, rowsbuf = pbufs[b], colbs[b], rowsbufs[b]
            npbuf, ncolb, nrowsbuf = pbufs[1 - b], colbs[1 - b], rowsbufs[1 - b]

            pltpu.make_async_copy(x_hbm.at[colb], rowsbuf, gsems[b]).wait()

            @pl.when(i + 1 < nb)
            def _():
                pltpu.make_async_copy(pk_slice(i + 1), npbuf,
                                      isems[1 - b]).wait()
                _extract_colb(npbuf, ncolb)
                pltpu.async_copy(x_hbm.at[ncolb], nrowsbuf, gsems[1 - b])

            for j8 in range(B // 16):
                j16 = j8 * 16 + _iota16()
                r16 = plsc.load_gather(pbuf, [j16 * 3 + 1])
                l16 = r16 - base
                bad = (l16 < 0) | (l16 >= PASS_R)
                lidxb[pl.ds(j8 * 16, 16)] = jnp.where(bad, TRASH, l16)

            @functools.partial(lax.fori_loop, 0, B // 2, init_val=0)
            def _(j2, scarry):
                for u in range(2):
                    jv = jnp.full((16,), j2 * 2 + u, dtype=jnp.int32)
                    v = plsc.bitcast(plsc.load_gather(pbuf, [jv * 3 + 2]),
                                     jnp.float32)
                    for d in range(D // 16):
                        ci = d * 16 + _iota16()
                        xv = plsc.load_gather(rowsbuf, [jv, ci])
                        plsc.store_scatter(rowsbuf, [jv, ci], xv * v)
                return scarry

            pltpu.sync_copy(rowsbuf, acc.at[lidxb], add=True)

            @pl.when(i + 2 < nb)
            def _():
                pltpu.async_copy(pk_slice(i + 2), pbuf, isems[b])

        def pair_body(i2, bcarry):
            for b in range(2):
                i = i2 * 2 + b

                @pl.when(i < nb)
                def _():
                    slot(i, b)
            return bcarry

        lax.fori_loop(0, (nb + 1) // 2, pair_body, 0)

        plsc.subcore_barrier()

        @pl.when(s < NS - 1)
        def _():
            pltpu.sync_copy(acc.at[pl.ds(zlo, WSLICE)],
                            out_hbm.at[pl.ds(base + zlo, WSLICE)])

        @pl.when(s == NS - 1)
        def _():
            pltpu.sync_copy(acc.at[pl.ds(zlo, W15)],
                            out_hbm.at[pl.ds(base + zlo, W15)])

        return carry

    lax.fori_loop(0, NPASS, pass_body, 0)


_spmm_call = pl.kernel(
    _spmm_body,
    out_type=jax.ShapeDtypeStruct((N_TOT, D), jnp.float32),
    mesh=plsc.VectorSubcoreMesh(core_axis_name="c", subcore_axis_name="s",
                                num_cores=NC, num_subcores=NS),
    compiler_params=pltpu.CompilerParams(needs_layout_passes=False),
    scratch_types=[
        pltpu.VMEM_SHARED((ACC_ROWS, D), jnp.float32),
        pltpu.VMEM((3 * B,), jnp.int32),
        pltpu.VMEM((3 * B,), jnp.int32),
        pltpu.VMEM((B,), jnp.int32),
        pltpu.VMEM((B,), jnp.int32),
        pltpu.VMEM((B,), jnp.int32),
        pltpu.VMEM((B, D), jnp.float32),
        pltpu.VMEM((B, D), jnp.float32),
        pltpu.VMEM((16,), jnp.int32),
        pltpu.VMEM((16,), jnp.int32),
        pltpu.SemaphoreType.DMA,
        pltpu.SemaphoreType.DMA,
        pltpu.SemaphoreType.DMA,
        pltpu.SemaphoreType.DMA,
    ],
)


def _sort_edges(rows, cols, vals):
    order = jnp.argsort(rows)
    srows = jnp.take(rows, order)
    scols = jnp.take(cols, order)
    svals = jnp.take(vals, order)
    pad = EPAD - N_EDGES
    srows_p = jnp.concatenate([srows, jnp.full((pad,), jnp.int32(1 << 20))])
    scols_p = jnp.concatenate([scols, jnp.zeros((pad,), jnp.int32)])
    svals_p = jnp.concatenate([svals, jnp.zeros((pad,), jnp.float32)])
    vbits = lax.bitcast_convert_type(svals_p, jnp.int32)
    packed = jnp.stack([scols_p, srows_p, vbits], axis=1).reshape(-1)

    cuts = [min(c * N_ENT + p * PASS_R, (c + 1) * N_ENT)
            for c in range(NC) for p in range(NPASS)] + [N_TOT]
    bounds = jnp.searchsorted(srows, jnp.array(cuts, dtype=jnp.int32))
    starts = (bounds[:-1] // 8) * 8
    lens = bounds[1:] - starts
    nb = (lens + NS * B - 1) // (NS * B)
    starts_b = jnp.broadcast_to(
        starts[:, None], (NC * NPASS, 16)).astype(jnp.int32).reshape(-1)
    nb_b = jnp.broadcast_to(
        nb[:, None], (NC * NPASS, 16)).astype(jnp.int32).reshape(-1)
    return packed, starts_b, nb_b


def _gate_body(pre_ref, collab_ref, kg_ref, wb_ref, dual_ref):
    collab = collab_ref[...]
    kg = kg_ref[...]
    z = pre_ref[...] + jnp.dot(collab, wb_ref[...],
                               preferred_element_type=jnp.float32)
    g = jax.nn.sigmoid(z)
    dual_ref[...] = g * kg + (1.0 - g) * collab


def _gate(pre_a, collab, kg, W_b_t):
    grid = (N_ENT // GATE_BLK,)
    return pl.pallas_call(
        _gate_body,
        grid=grid,
        in_specs=[
            pl.BlockSpec((GATE_BLK, D), lambda i: (i, 0)),
            pl.BlockSpec((GATE_BLK, D), lambda i: (i, 0)),
            pl.BlockSpec((GATE_BLK, D), lambda i: (i, 0)),
            pl.BlockSpec((D, D), lambda i: (0, 0)),
        ],
        out_specs=pl.BlockSpec((GATE_BLK, D), lambda i: (i, 0)),
        out_shape=jax.ShapeDtypeStruct((N_ENT, D), jnp.float32),
    )(pre_a, collab, kg, W_b_t)


def _prea_body(ent_ref, wa_ref, out_ref):
    out_ref[...] = jnp.dot(ent_ref[...], wa_ref[...],
                           preferred_element_type=jnp.float32)


def _pre_a(e_entities, W_a_t):
    grid = (N_ENT // GATE_BLK,)
    return pl.pallas_call(
        _prea_body,
        grid=grid,
        in_specs=[
            pl.BlockSpec((GATE_BLK, D), lambda i: (i, 0)),
            pl.BlockSpec((D, D), lambda i: (0, 0)),
        ],
        out_specs=pl.BlockSpec((GATE_BLK, D), lambda i: (i, 0)),
        out_shape=jax.ShapeDtypeStruct((N_ENT, D), jnp.float32),
    )(e_entities, W_a_t)


def _score_body(u_ref, it_ref, out_ref):
    out_ref[...] = jnp.dot(u_ref[...], it_ref[...].T,
                           preferred_element_type=jnp.float32)


def _scores(user_embed, item_embed):
    return pl.pallas_call(
        _score_body,
        out_shape=jax.ShapeDtypeStruct((BATCH, BATCH), jnp.float32),
    )(user_embed, item_embed)


def kernel(entity_user_embed, W_a, W_b, A_vals, A_rows, A_cols, user_ids, item_ids):
    e_entities = entity_user_embed[:N_ENT]
    e_users = entity_user_embed[N_ENT:]
    W_a_t = W_a.T
    W_b_t = W_b.T

    pre_a = _pre_a(e_entities, W_a_t)
    packed, starts_b, nb_b = _sort_edges(A_rows, A_cols, A_vals)

    x = entity_user_embed
    collab_sum = e_entities
    users_sum = e_users
    for layer in range(3):
        y = _spmm_call(x, packed, starts_b, nb_b)
        collab = y[:N_ENT]
        users = y[N_ENT:]
        collab_sum = collab_sum + collab
        users_sum = users_sum + users
        if layer < 2:
            dual = _gate(pre_a, collab, e_entities, W_b_t)
            x = jnp.concatenate([dual, users], axis=0)

    all_embed = jnp.concatenate([collab_sum, users_sum], axis=0)
    user_embed = jnp.take(all_embed, user_ids, axis=0)
    item_embed = jnp.take(all_embed, item_ids, axis=0)
    return _scores(user_embed, item_embed)
